# baseline (speedup 1.0000x reference)
import functools
import jax
import jax.numpy as jnp
from jax.experimental import pallas as pl

EBLK = 512
NBLK = 512

_f32 = jnp.float32


def _iota(shape, dim):
    return jax.lax.broadcasted_iota(jnp.int32, shape, dim)


def _prep_body(enc_ref, spk_ref, emb_ref, out_ref):
    spk = spk_ref[0, 0, :]
    oh = (spk[:, None] == _iota((NBLK, emb_ref.shape[0]), 1)).astype(_f32)
    out_ref[:, :] = enc_ref[:, :] + jnp.dot(oh, emb_ref[:, :],
                                            preferred_element_type=_f32)


def _prep(enc_pad, spk3d, spk_emb):
    npad, d = enc_pad.shape
    nb = npad // NBLK
    return pl.pallas_call(
        _prep_body,
        grid=(nb,),
        in_specs=[
            pl.BlockSpec((NBLK, d), lambda i: (i, 0)),
            pl.BlockSpec((1, 1, NBLK), lambda i: (i, 0, 0)),
            pl.BlockSpec(spk_emb.shape, lambda i: (0, 0)),
        ],
        out_specs=pl.BlockSpec((NBLK, d), lambda i: (i, 0)),
        out_shape=jax.ShapeDtypeStruct((npad, d), _f32),
    )(enc_pad, spk3d, spk_emb)


def _deg_body(dst_ref, et1_ref, et3_ref, deg1_ref, deg3_ref, *, nblocks, r1, r3):
    ec = pl.program_id(0)

    @pl.when(ec == 0)
    def _init():
        deg1_ref[:, :] = jnp.zeros_like(deg1_ref)
        deg3_ref[:, :] = jnp.zeros_like(deg3_ref)

    d = dst_ref[0, 0, :]
    bf = jnp.bfloat16
    oh1 = (et1_ref[0, 0, :][:, None] == _iota((EBLK, r1), 1)).astype(bf)
    oh3 = (et3_ref[0, 0, :][:, None] == _iota((EBLK, r3), 1)).astype(bf)
    for m in range(nblocks):
        ohd = (d[None, :] == (m * NBLK + _iota((NBLK, EBLK), 0))).astype(bf)
        deg1_ref[pl.ds(m * NBLK, NBLK), :] += jnp.dot(
            ohd, oh1, preferred_element_type=_f32)
        deg3_ref[pl.ds(m * NBLK, NBLK), :] += jnp.dot(
            ohd, oh3, preferred_element_type=_f32)


def _degrees(dst3d, et1_3d, et3_3d, npad, r1, r3):
    nchunks = dst3d.shape[0]
    nblocks = npad // NBLK
    body = functools.partial(_deg_body, nblocks=nblocks, r1=r1, r3=r3)
    espec = pl.BlockSpec((1, 1, EBLK), lambda e: (e, 0, 0))
    return pl.pallas_call(
        body,
        grid=(nchunks,),
        in_specs=[espec, espec, espec],
        out_specs=[
            pl.BlockSpec((npad, r1), lambda e: (0, 0)),
            pl.BlockSpec((npad, r3), lambda e: (0, 0)),
        ],
        out_shape=[
            jax.ShapeDtypeStruct((npad, r1), _f32),
            jax.ShapeDtypeStruct((npad, r3), _f32),
        ],
    )(dst3d, et1_3d, et3_3d)


def _rgcn_body(x_ref, src_ref, dst_ref, ety_ref, deg_ref, bases_ref, comp_ref,
               root_ref, bias_ref, out_ref, *, nblocks, r, nb_bases):
    ec = pl.program_id(0)

    @pl.when(ec == 0)
    def _init():
        out_ref[:, :] = jnp.dot(x_ref[:, :], root_ref[:, :],
                                preferred_element_type=_f32) + bias_ref[0, :]

    s = src_ref[0, 0, :]
    d = dst_ref[0, 0, :]
    bf = jnp.bfloat16
    oh_t = (ety_ref[0, 0, :][:, None] == _iota((EBLK, r), 1)).astype(_f32)
    xc = x_ref[:, :].astype(bf)

    g = jnp.zeros((EBLK, x_ref.shape[1]), _f32)
    for m in range(nblocks):
        ohs = (s[:, None] == (m * NBLK + _iota((EBLK, NBLK), 1))).astype(bf)
        g = g + jnp.dot(ohs, xc[m * NBLK:(m + 1) * NBLK, :],
                        preferred_element_type=_f32)

    c = jnp.dot(oh_t, comp_ref[:, :], preferred_element_type=_f32)
    msg = jnp.zeros_like(g)
    for b in range(nb_bases):
        msg = msg + c[:, b:b + 1] * jnp.dot(g, bases_ref[b],
                                            preferred_element_type=_f32)
    msgc = msg.astype(bf)

    for m in range(nblocks):
        ohd = (d[None, :] == (m * NBLK + _iota((NBLK, EBLK), 0))).astype(_f32)
        dinv = 1.0 / jnp.maximum(deg_ref[pl.ds(m * NBLK, NBLK), :], 1.0)
        w = jnp.dot(dinv, oh_t.T, preferred_element_type=_f32)
        out_ref[pl.ds(m * NBLK, NBLK), :] += jnp.dot(
            (ohd * w).astype(bf), msgc, preferred_element_type=_f32)


def _rgcn_layer(x, src3d, dst3d, ety3d, deg, bases, comp, root, bias):
    npad, dim = x.shape
    nchunks = src3d.shape[0]
    nblocks = npad // NBLK
    r, nb_bases = comp.shape
    body = functools.partial(_rgcn_body, nblocks=nblocks, r=r,
                             nb_bases=nb_bases)
    espec = pl.BlockSpec((1, 1, EBLK), lambda e: (e, 0, 0))
    whole = lambda shp: pl.BlockSpec(shp, lambda e: tuple(0 for _ in shp))
    return pl.pallas_call(
        body,
        grid=(nchunks,),
        in_specs=[
            whole((npad, dim)), espec, espec, espec,
            whole((npad, r)), whole(bases.shape), whole((r, nb_bases)),
            whole((dim, dim)), whole((1, dim)),
        ],
        out_specs=pl.BlockSpec((npad, dim), lambda e: (0, 0)),
        out_shape=jax.ShapeDtypeStruct((npad, dim), _f32),
    )(x, src3d, dst3d, ety3d, deg, bases, comp, root, bias.reshape(1, -1))


def _gconv_body(x_ref, src_ref, dst_ref, wrel_ref, brel_ref, wroot_ref,
                out_ref, *, nblocks):
    ec = pl.program_id(0)

    @pl.when(ec == 0)
    def _init():
        out_ref[:, :] = jnp.dot(x_ref[:, :], wroot_ref[:, :],
                                preferred_element_type=_f32) + brel_ref[0, :]

    s = src_ref[0, 0, :]
    d = dst_ref[0, 0, :]
    bf = jnp.bfloat16
    xc = x_ref[:, :].astype(bf)
    g = jnp.zeros((EBLK, x_ref.shape[1]), _f32)
    for m in range(nblocks):
        ohs = (s[:, None] == (m * NBLK + _iota((EBLK, NBLK), 1))).astype(bf)
        g = g + jnp.dot(ohs, xc[m * NBLK:(m + 1) * NBLK, :],
                        preferred_element_type=_f32)
    msgc = jnp.dot(g, wrel_ref[:, :],
                   preferred_element_type=_f32).astype(bf)
    for m in range(nblocks):
        ohd = (d[None, :] == (m * NBLK + _iota((NBLK, EBLK), 0))).astype(bf)
        out_ref[pl.ds(m * NBLK, NBLK), :] += jnp.dot(
            ohd, msgc, preferred_element_type=_f32)


def _gconv_layer(x, src3d, dst3d, wrel, brel, wroot):
    npad, dim = x.shape
    nchunks = src3d.shape[0]
    nblocks = npad // NBLK
    body = functools.partial(_gconv_body, nblocks=nblocks)
    espec = pl.BlockSpec((1, 1, EBLK), lambda e: (e, 0, 0))
    whole = lambda shp: pl.BlockSpec(shp, lambda e: tuple(0 for _ in shp))
    return pl.pallas_call(
        body,
        grid=(nchunks,),
        in_specs=[
            whole((npad, dim)), espec, espec,
            whole((dim, dim)), whole((1, dim)), whole((dim, dim)),
        ],
        out_specs=pl.BlockSpec((npad, dim), lambda e: (0, 0)),
        out_shape=jax.ShapeDtypeStruct((npad, dim), _f32),
    )(x, src3d, dst3d, wrel, brel.reshape(1, -1), wroot)


def _dec_body(enc_ref, u1_ref, u2_ref, w0a_ref, w0b_ref, w0c_ref, b0_ref,
              w1_ref, b1_ref, w2_ref, b2_ref, out_ref):
    h = (jnp.dot(enc_ref[:, :], w0a_ref[:, :], preferred_element_type=_f32)
         + jnp.dot(u1_ref[:, :], w0b_ref[:, :], preferred_element_type=_f32)
         + jnp.dot(u2_ref[:, :], w0c_ref[:, :], preferred_element_type=_f32)
         + b0_ref[0, :])
    h = jnp.maximum(h, 0.0)
    h = jnp.maximum(jnp.dot(h, w1_ref[:, :], preferred_element_type=_f32)
                    + b1_ref[0, :], 0.0)
    res = jnp.dot(h, w2_ref[:, :], preferred_element_type=_f32) + b2_ref[0, :]
    out_ref[:, :] = jnp.broadcast_to(res, out_ref.shape)


def _decoder(enc_pad, u1, u2, w0, b0, w1, b1, w2, b2):
    npad, dim = enc_pad.shape
    nb = npad // NBLK
    h1 = w0.shape[1]
    h2 = w1.shape[1]
    w0a, w0b, w0c = w0[:dim], w0[dim:2 * dim], w0[2 * dim:]
    whole = lambda shp: pl.BlockSpec(shp, lambda i: tuple(0 for _ in shp))
    nspec = pl.BlockSpec((NBLK, dim), lambda i: (i, 0))
    return pl.pallas_call(
        _dec_body,
        grid=(nb,),
        in_specs=[
            nspec, nspec, nspec,
            whole((dim, h1)), whole((dim, h1)), whole((dim, h1)),
            whole((1, h1)),
            whole((h1, h2)), whole((1, h2)),
            whole((h2, 1)), whole((1, 1)),
        ],
        out_specs=pl.BlockSpec((NBLK, 128), lambda i: (i, 0)),
        out_shape=jax.ShapeDtypeStruct((npad, 128), _f32),
    )(enc_pad, u1, u2, w0a, w0b, w0c, b0.reshape(1, -1),
      w1, b1.reshape(1, -1), w2, b2.reshape(1, -1))


def kernel(encoding, speaker, edge_index, edge_type, edge_speaker_type,
           spk_emb, c1_bases, c1_comp, c1_root, c1_bias,
           c2_wrel, c2_brel, c2_wroot,
           c3_bases, c3_comp, c3_root, c3_bias,
           c4_wrel, c4_brel, c4_wroot,
           dec_w0, dec_b0, dec_w1, dec_b1, dec_w2, dec_b2):
    n, dim = encoding.shape
    e = edge_index.shape[1]
    npad = ((n + NBLK - 1) // NBLK) * NBLK
    assert e % EBLK == 0
    nchunks = e // EBLK

    enc_pad = jnp.pad(encoding, ((0, npad - n), (0, 0)))
    spk3d = jnp.pad(speaker.astype(jnp.int32),
                    (0, npad - n)).reshape(npad // NBLK, 1, NBLK)
    src3d = edge_index[0].astype(jnp.int32).reshape(nchunks, 1, EBLK)
    dst3d = edge_index[1].astype(jnp.int32).reshape(nchunks, 1, EBLK)
    et1 = edge_type.astype(jnp.int32).reshape(nchunks, 1, EBLK)
    et3 = edge_speaker_type.astype(jnp.int32).reshape(nchunks, 1, EBLK)

    utt = _prep(enc_pad, spk3d, spk_emb)
    deg1, deg3 = _degrees(dst3d, et1, et3, npad,
                          c1_comp.shape[0], c3_comp.shape[0])

    u1 = _rgcn_layer(utt, src3d, dst3d, et1, deg1,
                     c1_bases, c1_comp, c1_root, c1_bias)
    u1 = _gconv_layer(u1, src3d, dst3d, c2_wrel, c2_brel, c2_wroot)
    u2 = _rgcn_layer(utt, src3d, dst3d, et3, deg3,
                     c3_bases, c3_comp, c3_root, c3_bias)
    u2 = _gconv_layer(u2, src3d, dst3d, c4_wrel, c4_brel, c4_wroot)

    out = _decoder(enc_pad, u1, u2, dec_w0, dec_b0, dec_w1, dec_b1,
                   dec_w2, dec_b2)
    return out[:n, 0]


# speedup vs baseline: 1.5581x; 1.5581x over previous
import functools
import jax
import jax.numpy as jnp
from jax.experimental import pallas as pl

EBLK = 512
NBLK = 512

_f32 = jnp.float32


def _iota(shape, dim):
    return jax.lax.broadcasted_iota(jnp.int32, shape, dim)


def _prep_body(enc_ref, spk_ref, emb_ref, out_ref):
    spk = spk_ref[0, 0, :]
    oh = (spk[:, None] == _iota((NBLK, emb_ref.shape[0]), 1)).astype(_f32)
    out_ref[:, :] = enc_ref[:, :] + jnp.dot(oh, emb_ref[:, :],
                                            preferred_element_type=_f32)


def _prep(enc_pad, spk3d, spk_emb):
    npad, d = enc_pad.shape
    nb = npad // NBLK
    return pl.pallas_call(
        _prep_body,
        grid=(nb,),
        in_specs=[
            pl.BlockSpec((NBLK, d), lambda i: (i, 0)),
            pl.BlockSpec((1, 1, NBLK), lambda i: (i, 0, 0)),
            pl.BlockSpec(spk_emb.shape, lambda i: (0, 0)),
        ],
        out_specs=pl.BlockSpec((NBLK, d), lambda i: (i, 0)),
        out_shape=jax.ShapeDtypeStruct((npad, d), _f32),
    )(enc_pad, spk3d, spk_emb)


def _deg_body(dst_ref, et1_ref, et3_ref, deg1_ref, deg3_ref, *, nblocks, r1, r3):
    ec = pl.program_id(0)

    @pl.when(ec == 0)
    def _init():
        deg1_ref[:, :] = jnp.zeros_like(deg1_ref)
        deg3_ref[:, :] = jnp.zeros_like(deg3_ref)

    d = dst_ref[0, 0, :]
    bf = jnp.bfloat16
    oh1 = (et1_ref[0, 0, :][:, None] == _iota((EBLK, r1), 1)).astype(_f32)
    oh3 = (et3_ref[0, 0, :][:, None] == _iota((EBLK, r3), 1)).astype(_f32)
    lo = d % NBLK
    hi = d // NBLK
    ld = (lo[None, :] == _iota((NBLK, EBLK), 0)).astype(bf)
    for m in range(nblocks):
        hm = (hi == m).astype(_f32)[:, None]
        deg1_ref[pl.ds(m * NBLK, NBLK), :] += jnp.dot(
            ld, (oh1 * hm).astype(bf), preferred_element_type=_f32)
        deg3_ref[pl.ds(m * NBLK, NBLK), :] += jnp.dot(
            ld, (oh3 * hm).astype(bf), preferred_element_type=_f32)


def _degrees(dst3d, et1_3d, et3_3d, npad, r1, r3):
    nchunks = dst3d.shape[0]
    nblocks = npad // NBLK
    body = functools.partial(_deg_body, nblocks=nblocks, r1=r1, r3=r3)
    espec = pl.BlockSpec((1, 1, EBLK), lambda e: (e, 0, 0))
    return pl.pallas_call(
        body,
        grid=(nchunks,),
        in_specs=[espec, espec, espec],
        out_specs=[
            pl.BlockSpec((npad, r1), lambda e: (0, 0)),
            pl.BlockSpec((npad, r3), lambda e: (0, 0)),
        ],
        out_shape=[
            jax.ShapeDtypeStruct((npad, r1), _f32),
            jax.ShapeDtypeStruct((npad, r3), _f32),
        ],
    )(dst3d, et1_3d, et3_3d)


def _rgcn_body(x_ref, src_ref, dst_ref, ety_ref, deg_ref, bases_ref, comp_ref,
               root_ref, bias_ref, out_ref, *, nblocks, r, nb_bases):
    ec = pl.program_id(0)

    @pl.when(ec == 0)
    def _init():
        out_ref[:, :] = jnp.dot(x_ref[:, :], root_ref[:, :],
                                preferred_element_type=_f32) + bias_ref[0, :]

    s = src_ref[0, 0, :]
    d = dst_ref[0, 0, :]
    bf = jnp.bfloat16
    oh_t = (ety_ref[0, 0, :][:, None] == _iota((EBLK, r), 1)).astype(_f32)
    xc = x_ref[:, :].astype(bf)

    lo_s = s % NBLK
    hi_s = s // NBLK
    ls = (lo_s[:, None] == _iota((EBLK, NBLK), 1)).astype(bf)
    g = jnp.zeros((EBLK, x_ref.shape[1]), _f32)
    for m in range(nblocks):
        hm = (hi_s == m).astype(_f32)[:, None]
        t = jnp.dot(ls, xc[m * NBLK:(m + 1) * NBLK, :],
                    preferred_element_type=_f32)
        g = g + hm * t

    c = jnp.dot(oh_t, comp_ref[:, :], preferred_element_type=_f32)
    msg = jnp.zeros_like(g)
    for b in range(nb_bases):
        msg = msg + c[:, b:b + 1] * jnp.dot(g, bases_ref[b],
                                            preferred_element_type=_f32)

    lo_d = d % NBLK
    hi_d = d // NBLK
    ld = (lo_d[None, :] == _iota((NBLK, EBLK), 0)).astype(bf)
    rows = jnp.zeros((EBLK, oh_t.shape[1]), _f32)
    for m in range(nblocks):
        hm = (hi_d == m).astype(_f32)[:, None]
        dinv = (1.0 / jnp.maximum(deg_ref[pl.ds(m * NBLK, NBLK), :],
                                  1.0)).astype(bf)
        rows = rows + hm * jnp.dot(ld.T, dinv, preferred_element_type=_f32)
    norm = jnp.sum(rows * oh_t, axis=1, keepdims=True)
    msgw = msg * norm

    for m in range(nblocks):
        hm = (hi_d == m).astype(_f32)[:, None]
        out_ref[pl.ds(m * NBLK, NBLK), :] += jnp.dot(
            ld, (msgw * hm).astype(bf), preferred_element_type=_f32)


def _rgcn_layer(x, src3d, dst3d, ety3d, deg, bases, comp, root, bias):
    npad, dim = x.shape
    nchunks = src3d.shape[0]
    nblocks = npad // NBLK
    r, nb_bases = comp.shape
    body = functools.partial(_rgcn_body, nblocks=nblocks, r=r,
                             nb_bases=nb_bases)
    espec = pl.BlockSpec((1, 1, EBLK), lambda e: (e, 0, 0))
    whole = lambda shp: pl.BlockSpec(shp, lambda e: tuple(0 for _ in shp))
    return pl.pallas_call(
        body,
        grid=(nchunks,),
        in_specs=[
            whole((npad, dim)), espec, espec, espec,
            whole((npad, r)), whole(bases.shape), whole((r, nb_bases)),
            whole((dim, dim)), whole((1, dim)),
        ],
        out_specs=pl.BlockSpec((npad, dim), lambda e: (0, 0)),
        out_shape=jax.ShapeDtypeStruct((npad, dim), _f32),
    )(x, src3d, dst3d, ety3d, deg, bases, comp, root, bias.reshape(1, -1))


def _gconv_body(x_ref, src_ref, dst_ref, wrel_ref, brel_ref, wroot_ref,
                out_ref, *, nblocks):
    ec = pl.program_id(0)

    @pl.when(ec == 0)
    def _init():
        out_ref[:, :] = jnp.dot(x_ref[:, :], wroot_ref[:, :],
                                preferred_element_type=_f32) + brel_ref[0, :]

    s = src_ref[0, 0, :]
    d = dst_ref[0, 0, :]
    bf = jnp.bfloat16
    xc = x_ref[:, :].astype(bf)
    lo_s = s % NBLK
    hi_s = s // NBLK
    ls = (lo_s[:, None] == _iota((EBLK, NBLK), 1)).astype(bf)
    g = jnp.zeros((EBLK, x_ref.shape[1]), _f32)
    for m in range(nblocks):
        hm = (hi_s == m).astype(_f32)[:, None]
        t = jnp.dot(ls, xc[m * NBLK:(m + 1) * NBLK, :],
                    preferred_element_type=_f32)
        g = g + hm * t
    msg = jnp.dot(g, wrel_ref[:, :], preferred_element_type=_f32)
    lo_d = d % NBLK
    hi_d = d // NBLK
    ld = (lo_d[None, :] == _iota((NBLK, EBLK), 0)).astype(bf)
    for m in range(nblocks):
        hm = (hi_d == m).astype(_f32)[:, None]
        out_ref[pl.ds(m * NBLK, NBLK), :] += jnp.dot(
            ld, (msg * hm).astype(bf), preferred_element_type=_f32)


def _gconv_layer(x, src3d, dst3d, wrel, brel, wroot):
    npad, dim = x.shape
    nchunks = src3d.shape[0]
    nblocks = npad // NBLK
    body = functools.partial(_gconv_body, nblocks=nblocks)
    espec = pl.BlockSpec((1, 1, EBLK), lambda e: (e, 0, 0))
    whole = lambda shp: pl.BlockSpec(shp, lambda e: tuple(0 for _ in shp))
    return pl.pallas_call(
        body,
        grid=(nchunks,),
        in_specs=[
            whole((npad, dim)), espec, espec,
            whole((dim, dim)), whole((1, dim)), whole((dim, dim)),
        ],
        out_specs=pl.BlockSpec((npad, dim), lambda e: (0, 0)),
        out_shape=jax.ShapeDtypeStruct((npad, dim), _f32),
    )(x, src3d, dst3d, wrel, brel.reshape(1, -1), wroot)


def _dec_body(enc_ref, u1_ref, u2_ref, w0a_ref, w0b_ref, w0c_ref, b0_ref,
              w1_ref, b1_ref, w2_ref, b2_ref, out_ref):
    h = (jnp.dot(enc_ref[:, :], w0a_ref[:, :], preferred_element_type=_f32)
         + jnp.dot(u1_ref[:, :], w0b_ref[:, :], preferred_element_type=_f32)
         + jnp.dot(u2_ref[:, :], w0c_ref[:, :], preferred_element_type=_f32)
         + b0_ref[0, :])
    h = jnp.maximum(h, 0.0)
    h = jnp.maximum(jnp.dot(h, w1_ref[:, :], preferred_element_type=_f32)
                    + b1_ref[0, :], 0.0)
    res = jnp.dot(h, w2_ref[:, :], preferred_element_type=_f32) + b2_ref[0, :]
    out_ref[:, :] = jnp.broadcast_to(res, out_ref.shape)


def _decoder(enc_pad, u1, u2, w0, b0, w1, b1, w2, b2):
    npad, dim = enc_pad.shape
    nb = npad // NBLK
    h1 = w0.shape[1]
    h2 = w1.shape[1]
    w0a, w0b, w0c = w0[:dim], w0[dim:2 * dim], w0[2 * dim:]
    whole = lambda shp: pl.BlockSpec(shp, lambda i: tuple(0 for _ in shp))
    nspec = pl.BlockSpec((NBLK, dim), lambda i: (i, 0))
    return pl.pallas_call(
        _dec_body,
        grid=(nb,),
        in_specs=[
            nspec, nspec, nspec,
            whole((dim, h1)), whole((dim, h1)), whole((dim, h1)),
            whole((1, h1)),
            whole((h1, h2)), whole((1, h2)),
            whole((h2, 1)), whole((1, 1)),
        ],
        out_specs=pl.BlockSpec((NBLK, 128), lambda i: (i, 0)),
        out_shape=jax.ShapeDtypeStruct((npad, 128), _f32),
    )(enc_pad, u1, u2, w0a, w0b, w0c, b0.reshape(1, -1),
      w1, b1.reshape(1, -1), w2, b2.reshape(1, -1))


def kernel(encoding, speaker, edge_index, edge_type, edge_speaker_type,
           spk_emb, c1_bases, c1_comp, c1_root, c1_bias,
           c2_wrel, c2_brel, c2_wroot,
           c3_bases, c3_comp, c3_root, c3_bias,
           c4_wrel, c4_brel, c4_wroot,
           dec_w0, dec_b0, dec_w1, dec_b1, dec_w2, dec_b2):
    n, dim = encoding.shape
    e = edge_index.shape[1]
    npad = ((n + NBLK - 1) // NBLK) * NBLK
    assert e % EBLK == 0
    nchunks = e // EBLK

    enc_pad = jnp.pad(encoding, ((0, npad - n), (0, 0)))
    spk3d = jnp.pad(speaker.astype(jnp.int32),
                    (0, npad - n)).reshape(npad // NBLK, 1, NBLK)
    src3d = edge_index[0].astype(jnp.int32).reshape(nchunks, 1, EBLK)
    dst3d = edge_index[1].astype(jnp.int32).reshape(nchunks, 1, EBLK)
    et1 = edge_type.astype(jnp.int32).reshape(nchunks, 1, EBLK)
    et3 = edge_speaker_type.astype(jnp.int32).reshape(nchunks, 1, EBLK)

    utt = _prep(enc_pad, spk3d, spk_emb)
    deg1, deg3 = _degrees(dst3d, et1, et3, npad,
                          c1_comp.shape[0], c3_comp.shape[0])

    u1 = _rgcn_layer(utt, src3d, dst3d, et1, deg1,
                     c1_bases, c1_comp, c1_root, c1_bias)
    u1 = _gconv_layer(u1, src3d, dst3d, c2_wrel, c2_brel, c2_wroot)
    u2 = _rgcn_layer(utt, src3d, dst3d, et3, deg3,
                     c3_bases, c3_comp, c3_root, c3_bias)
    u2 = _gconv_layer(u2, src3d, dst3d, c4_wrel, c4_brel, c4_wroot)

    out = _decoder(enc_pad, u1, u2, dec_w0, dec_b0, dec_w1, dec_b1,
                   dec_w2, dec_b2)
    return out[:n, 0]


# speedup vs baseline: 1.6281x; 1.0450x over previous
import functools
import jax
import jax.numpy as jnp
from jax.experimental import pallas as pl

EBLK = 1280
NBLK = 1024

_f32 = jnp.float32


def _iota(shape, dim):
    return jax.lax.broadcasted_iota(jnp.int32, shape, dim)


def _prep_body(enc_ref, spk_ref, emb_ref, out_ref):
    spk = spk_ref[0, 0, :]
    oh = (spk[:, None] == _iota((NBLK, emb_ref.shape[0]), 1)).astype(_f32)
    out_ref[:, :] = enc_ref[:, :] + jnp.dot(oh, emb_ref[:, :],
                                            preferred_element_type=_f32)


def _prep(enc_pad, spk3d, spk_emb):
    npad, d = enc_pad.shape
    nb = npad // NBLK
    return pl.pallas_call(
        _prep_body,
        grid=(nb,),
        in_specs=[
            pl.BlockSpec((NBLK, d), lambda i: (i, 0)),
            pl.BlockSpec((1, 1, NBLK), lambda i: (i, 0, 0)),
            pl.BlockSpec(spk_emb.shape, lambda i: (0, 0)),
        ],
        out_specs=pl.BlockSpec((NBLK, d), lambda i: (i, 0)),
        out_shape=jax.ShapeDtypeStruct((npad, d), _f32),
    )(enc_pad, spk3d, spk_emb)


def _deg_body(dst_ref, et1_ref, et3_ref, deg1_ref, deg3_ref, *, nblocks, r1, r3):
    ec = pl.program_id(0)

    @pl.when(ec == 0)
    def _init():
        deg1_ref[:, :] = jnp.zeros_like(deg1_ref)
        deg3_ref[:, :] = jnp.zeros_like(deg3_ref)

    d = dst_ref[0, 0, :]
    bf = jnp.bfloat16
    oh1 = (et1_ref[0, 0, :][:, None] == _iota((EBLK, r1), 1)).astype(_f32)
    oh3 = (et3_ref[0, 0, :][:, None] == _iota((EBLK, r3), 1)).astype(_f32)
    lo = d % NBLK
    hi = d // NBLK
    ld = (lo[None, :] == _iota((NBLK, EBLK), 0)).astype(bf)
    for m in range(nblocks):
        hm = (hi == m).astype(_f32)[:, None]
        deg1_ref[pl.ds(m * NBLK, NBLK), :] += jnp.dot(
            ld, (oh1 * hm).astype(bf), preferred_element_type=_f32)
        deg3_ref[pl.ds(m * NBLK, NBLK), :] += jnp.dot(
            ld, (oh3 * hm).astype(bf), preferred_element_type=_f32)


def _degrees(dst3d, et1_3d, et3_3d, npad, r1, r3):
    nchunks = dst3d.shape[0]
    nblocks = npad // NBLK
    body = functools.partial(_deg_body, nblocks=nblocks, r1=r1, r3=r3)
    espec = pl.BlockSpec((1, 1, EBLK), lambda e: (e, 0, 0))
    return pl.pallas_call(
        body,
        grid=(nchunks,),
        in_specs=[espec, espec, espec],
        out_specs=[
            pl.BlockSpec((npad, r1), lambda e: (0, 0)),
            pl.BlockSpec((npad, r3), lambda e: (0, 0)),
        ],
        out_shape=[
            jax.ShapeDtypeStruct((npad, r1), _f32),
            jax.ShapeDtypeStruct((npad, r3), _f32),
        ],
    )(dst3d, et1_3d, et3_3d)


def _rgcn_body(x_ref, src_ref, dst_ref, ety_ref, deg_ref, bases_ref, comp_ref,
               root_ref, bias_ref, out_ref, *, nblocks, r, nb_bases):
    ec = pl.program_id(0)

    @pl.when(ec == 0)
    def _init():
        out_ref[:, :] = jnp.dot(x_ref[:, :], root_ref[:, :],
                                preferred_element_type=_f32) + bias_ref[0, :]

    s = src_ref[0, 0, :]
    d = dst_ref[0, 0, :]
    bf = jnp.bfloat16
    oh_t = (ety_ref[0, 0, :][:, None] == _iota((EBLK, r), 1)).astype(_f32)
    xc = x_ref[:, :].astype(bf)

    lo_s = s % NBLK
    hi_s = s // NBLK
    ls = (lo_s[:, None] == _iota((EBLK, NBLK), 1)).astype(bf)
    g = jnp.zeros((EBLK, x_ref.shape[1]), _f32)
    for m in range(nblocks):
        hm = (hi_s == m).astype(_f32)[:, None]
        t = jnp.dot(ls, xc[m * NBLK:(m + 1) * NBLK, :],
                    preferred_element_type=_f32)
        g = g + hm * t

    c = jnp.dot(oh_t, comp_ref[:, :], preferred_element_type=_f32)
    msg = jnp.zeros_like(g)
    for b in range(nb_bases):
        msg = msg + c[:, b:b + 1] * jnp.dot(g, bases_ref[b],
                                            preferred_element_type=_f32)

    lo_d = d % NBLK
    hi_d = d // NBLK
    ld = (lo_d[None, :] == _iota((NBLK, EBLK), 0)).astype(bf)
    rows = jnp.zeros((EBLK, oh_t.shape[1]), _f32)
    for m in range(nblocks):
        hm = (hi_d == m).astype(_f32)[:, None]
        dinv = (1.0 / jnp.maximum(deg_ref[pl.ds(m * NBLK, NBLK), :],
                                  1.0)).astype(bf)
        rows = rows + hm * jnp.dot(ld.T, dinv, preferred_element_type=_f32)
    norm = jnp.sum(rows * oh_t, axis=1, keepdims=True)
    msgw = msg * norm

    for m in range(nblocks):
        hm = (hi_d == m).astype(_f32)[:, None]
        out_ref[pl.ds(m * NBLK, NBLK), :] += jnp.dot(
            ld, (msgw * hm).astype(bf), preferred_element_type=_f32)


def _rgcn_layer(x, src3d, dst3d, ety3d, deg, bases, comp, root, bias):
    npad, dim = x.shape
    nchunks = src3d.shape[0]
    nblocks = npad // NBLK
    r, nb_bases = comp.shape
    body = functools.partial(_rgcn_body, nblocks=nblocks, r=r,
                             nb_bases=nb_bases)
    espec = pl.BlockSpec((1, 1, EBLK), lambda e: (e, 0, 0))
    whole = lambda shp: pl.BlockSpec(shp, lambda e: tuple(0 for _ in shp))
    return pl.pallas_call(
        body,
        grid=(nchunks,),
        in_specs=[
            whole((npad, dim)), espec, espec, espec,
            whole((npad, r)), whole(bases.shape), whole((r, nb_bases)),
            whole((dim, dim)), whole((1, dim)),
        ],
        out_specs=pl.BlockSpec((npad, dim), lambda e: (0, 0)),
        out_shape=jax.ShapeDtypeStruct((npad, dim), _f32),
    )(x, src3d, dst3d, ety3d, deg, bases, comp, root, bias.reshape(1, -1))


def _gconv_body(x_ref, src_ref, dst_ref, wrel_ref, brel_ref, wroot_ref,
                out_ref, *, nblocks):
    ec = pl.program_id(0)

    @pl.when(ec == 0)
    def _init():
        out_ref[:, :] = jnp.dot(x_ref[:, :], wroot_ref[:, :],
                                preferred_element_type=_f32) + brel_ref[0, :]

    s = src_ref[0, 0, :]
    d = dst_ref[0, 0, :]
    bf = jnp.bfloat16
    xc = x_ref[:, :].astype(bf)
    lo_s = s % NBLK
    hi_s = s // NBLK
    ls = (lo_s[:, None] == _iota((EBLK, NBLK), 1)).astype(bf)
    g = jnp.zeros((EBLK, x_ref.shape[1]), _f32)
    for m in range(nblocks):
        hm = (hi_s == m).astype(_f32)[:, None]
        t = jnp.dot(ls, xc[m * NBLK:(m + 1) * NBLK, :],
                    preferred_element_type=_f32)
        g = g + hm * t
    msg = jnp.dot(g, wrel_ref[:, :], preferred_element_type=_f32)
    lo_d = d % NBLK
    hi_d = d // NBLK
    ld = (lo_d[None, :] == _iota((NBLK, EBLK), 0)).astype(bf)
    for m in range(nblocks):
        hm = (hi_d == m).astype(_f32)[:, None]
        out_ref[pl.ds(m * NBLK, NBLK), :] += jnp.dot(
            ld, (msg * hm).astype(bf), preferred_element_type=_f32)


def _gconv_layer(x, src3d, dst3d, wrel, brel, wroot):
    npad, dim = x.shape
    nchunks = src3d.shape[0]
    nblocks = npad // NBLK
    body = functools.partial(_gconv_body, nblocks=nblocks)
    espec = pl.BlockSpec((1, 1, EBLK), lambda e: (e, 0, 0))
    whole = lambda shp: pl.BlockSpec(shp, lambda e: tuple(0 for _ in shp))
    return pl.pallas_call(
        body,
        grid=(nchunks,),
        in_specs=[
            whole((npad, dim)), espec, espec,
            whole((dim, dim)), whole((1, dim)), whole((dim, dim)),
        ],
        out_specs=pl.BlockSpec((npad, dim), lambda e: (0, 0)),
        out_shape=jax.ShapeDtypeStruct((npad, dim), _f32),
    )(x, src3d, dst3d, wrel, brel.reshape(1, -1), wroot)


def _dec_body(enc_ref, u1_ref, u2_ref, w0a_ref, w0b_ref, w0c_ref, b0_ref,
              w1_ref, b1_ref, w2_ref, b2_ref, out_ref):
    h = (jnp.dot(enc_ref[:, :], w0a_ref[:, :], preferred_element_type=_f32)
         + jnp.dot(u1_ref[:, :], w0b_ref[:, :], preferred_element_type=_f32)
         + jnp.dot(u2_ref[:, :], w0c_ref[:, :], preferred_element_type=_f32)
         + b0_ref[0, :])
    h = jnp.maximum(h, 0.0)
    h = jnp.maximum(jnp.dot(h, w1_ref[:, :], preferred_element_type=_f32)
                    + b1_ref[0, :], 0.0)
    res = jnp.dot(h, w2_ref[:, :], preferred_element_type=_f32) + b2_ref[0, :]
    out_ref[:, :] = jnp.broadcast_to(res, out_ref.shape)


def _decoder(enc_pad, u1, u2, w0, b0, w1, b1, w2, b2):
    npad, dim = enc_pad.shape
    nb = npad // NBLK
    h1 = w0.shape[1]
    h2 = w1.shape[1]
    w0a, w0b, w0c = w0[:dim], w0[dim:2 * dim], w0[2 * dim:]
    whole = lambda shp: pl.BlockSpec(shp, lambda i: tuple(0 for _ in shp))
    nspec = pl.BlockSpec((NBLK, dim), lambda i: (i, 0))
    return pl.pallas_call(
        _dec_body,
        grid=(nb,),
        in_specs=[
            nspec, nspec, nspec,
            whole((dim, h1)), whole((dim, h1)), whole((dim, h1)),
            whole((1, h1)),
            whole((h1, h2)), whole((1, h2)),
            whole((h2, 1)), whole((1, 1)),
        ],
        out_specs=pl.BlockSpec((NBLK, 128), lambda i: (i, 0)),
        out_shape=jax.ShapeDtypeStruct((npad, 128), _f32),
    )(enc_pad, u1, u2, w0a, w0b, w0c, b0.reshape(1, -1),
      w1, b1.reshape(1, -1), w2, b2.reshape(1, -1))


def kernel(encoding, speaker, edge_index, edge_type, edge_speaker_type,
           spk_emb, c1_bases, c1_comp, c1_root, c1_bias,
           c2_wrel, c2_brel, c2_wroot,
           c3_bases, c3_comp, c3_root, c3_bias,
           c4_wrel, c4_brel, c4_wroot,
           dec_w0, dec_b0, dec_w1, dec_b1, dec_w2, dec_b2):
    n, dim = encoding.shape
    e = edge_index.shape[1]
    npad = ((n + NBLK - 1) // NBLK) * NBLK
    assert e % EBLK == 0
    nchunks = e // EBLK

    enc_pad = jnp.pad(encoding, ((0, npad - n), (0, 0)))
    spk3d = jnp.pad(speaker.astype(jnp.int32),
                    (0, npad - n)).reshape(npad // NBLK, 1, NBLK)
    src3d = edge_index[0].astype(jnp.int32).reshape(nchunks, 1, EBLK)
    dst3d = edge_index[1].astype(jnp.int32).reshape(nchunks, 1, EBLK)
    et1 = edge_type.astype(jnp.int32).reshape(nchunks, 1, EBLK)
    et3 = edge_speaker_type.astype(jnp.int32).reshape(nchunks, 1, EBLK)

    utt = _prep(enc_pad, spk3d, spk_emb)
    deg1, deg3 = _degrees(dst3d, et1, et3, npad,
                          c1_comp.shape[0], c3_comp.shape[0])

    u1 = _rgcn_layer(utt, src3d, dst3d, et1, deg1,
                     c1_bases, c1_comp, c1_root, c1_bias)
    u1 = _gconv_layer(u1, src3d, dst3d, c2_wrel, c2_brel, c2_wroot)
    u2 = _rgcn_layer(utt, src3d, dst3d, et3, deg3,
                     c3_bases, c3_comp, c3_root, c3_bias)
    u2 = _gconv_layer(u2, src3d, dst3d, c4_wrel, c4_brel, c4_wroot)

    out = _decoder(enc_pad, u1, u2, dec_w0, dec_b0, dec_w1, dec_b1,
                   dec_w2, dec_b2)
    return out[:n, 0]
